# baseline (speedup 1.0000x reference)
import jax
import jax.numpy as jnp
from jax import lax
from jax.experimental import pallas as pl
from jax.experimental.pallas import tpu as pltpu
from jax.experimental.pallas import tpu_sc as plsc

N = 10000
NP = 10240
E = 320000
D = 128
H = 256
C = 6

NCORE = 2
NSUB = 16
CB = 125
NW = NCORE * NSUB
CPW = E // (NW * CB)
RPT = NP // NSUB
BM = 10240
GRID = NP // BM
STAGE = 16


def _sc_agg(table, ei4, with_deg, *, split_edges, stage):
    mesh = plsc.VectorSubcoreMesh(core_axis_name="c", subcore_axis_name="s")
    out_type = [jax.ShapeDtypeStruct((NCORE, NP, D), jnp.float32)]
    if with_deg:
        out_type.append(jax.ShapeDtypeStruct((NW, 1, NP), jnp.float32))
    spw = CPW // stage
    phases = spw if split_edges else 2 * spw
    scratch = [
        pltpu.VMEM_SHARED((NP, D), jnp.float32),
        pltpu.VMEM((stage, CB), jnp.int32),
        pltpu.VMEM((stage, CB), jnp.int32),
        pltpu.VMEM((CB, D), jnp.float32),
        pltpu.VMEM((CB, D), jnp.float32),
        pltpu.SemaphoreType.DMA,
        pltpu.SemaphoreType.DMA,
    ]
    if with_deg:
        scratch.append(pltpu.VMEM((NP,), jnp.float32))

    def body(*refs):
        if with_deg:
            (table_h, ei_h, agg_o, deg_o,
             acc, row_v, col_v, gbuf0, gbuf1, sem0, sem1, deg_v) = refs
        else:
            (table_h, ei_h, agg_o,
             acc, row_v, col_v, gbuf0, gbuf1, sem0, sem1) = refs
        c = lax.axis_index("c")
        s = lax.axis_index("s")
        tbl = table_h.at[c] if table.ndim == 3 else table_h

        zeros16 = jnp.zeros((16,), jnp.float32)

        def gzero(i, carry):
            gbuf0[i // 8, pl.ds((i % 8) * 16, 16)] = zeros16
            return carry

        lax.fori_loop(0, CB * 8, gzero, 0)
        for q in range(RPT // CB):
            pltpu.sync_copy(gbuf0, acc.at[pl.ds(s * RPT + q * CB, CB)])
        tail = RPT - (RPT // CB) * CB
        if tail:
            pltpu.sync_copy(gbuf0.at[pl.ds(0, tail)],
                            acc.at[pl.ds(s * RPT + (RPT // CB) * CB, tail)])

        if with_deg:
            def dzero(i, carry):
                deg_v[pl.ds(i * 16, 16)] = zeros16
                return carry

            lax.fori_loop(0, NP // 16, dzero, 0)

        plsc.subcore_barrier()

        ones16 = jnp.ones((16,), jnp.float32)
        tailmask = lax.iota(jnp.int32, 16) >= 3

        def dupd(j):
            if with_deg:
                for k in range(7):
                    idx = row_v[j, pl.ds(k * 16, 16)]
                    plsc.addupdate_scatter(deg_v, [idx], ones16)
                idx = row_v[j, pl.ds(CB - 16, 16)]
                plsc.addupdate_scatter(deg_v, [idx], ones16, mask=tailmask)

        for p in range(phases):
            if split_edges:
                wrow = c * NSUB + s
                roff = p * stage
            else:
                wrow = 2 * s + p // spw
                roff = (p % spw) * stage
            pltpu.sync_copy(ei_h.at[0, wrow, pl.ds(roff, stage)], row_v)
            pltpu.sync_copy(ei_h.at[1, wrow, pl.ds(roff, stage)], col_v)

            pltpu.async_copy(tbl.at[col_v.at[0]], gbuf0, sem0)

            def pair(i, carry):
                j0 = 2 * i
                pltpu.async_copy(tbl.at[col_v.at[j0 + 1]], gbuf1, sem1)
                pltpu.make_async_copy(
                    tbl.at[col_v.at[j0]], gbuf0, sem0).wait()
                dupd(j0)
                pltpu.sync_copy(gbuf0, acc.at[row_v.at[j0]], add=True)

                @pl.when(i < stage // 2 - 1)
                def _():
                    pltpu.async_copy(tbl.at[col_v.at[j0 + 2]], gbuf0, sem0)

                pltpu.make_async_copy(
                    tbl.at[col_v.at[j0 + 1]], gbuf1, sem1).wait()
                dupd(j0 + 1)
                pltpu.sync_copy(gbuf1, acc.at[row_v.at[j0 + 1]], add=True)
                return carry

            lax.fori_loop(0, stage // 2, pair, 0)

        if with_deg:
            w = c * NSUB + s
            pltpu.sync_copy(deg_v, deg_o.at[w, 0])

        plsc.subcore_barrier()
        pltpu.sync_copy(acc.at[pl.ds(s * RPT, RPT)],
                        agg_o.at[c, pl.ds(s * RPT, RPT)])

    return pl.kernel(
        body, out_type=out_type, mesh=mesh, scratch_types=scratch,
        compiler_params=pltpu.CompilerParams(needs_layout_passes=False),
    )(table, ei4)


_f32 = jnp.float32


def _dot(a, b):
    return jnp.dot(a, b, preferred_element_type=_f32)


def _wspec(*shape):
    nd = len(shape)
    return pl.BlockSpec(shape, lambda i, nd=nd: (0,) * nd)


def _rspec(*shape):
    return pl.BlockSpec(shape, lambda i: (i,) + (0,) * (len(shape) - 1))


_spec2 = pl.BlockSpec((2, BM, D), lambda i: (0, i, 0))
_dspec = pl.BlockSpec((NW, 1, BM), lambda i: (0, 0, i))


def _tc_pre1(x, sk0, b0r, sk1r, nk1r):
    def body(x_r, sk0_r, b0_r, sk1_r, nk1_r, s1a_r, t1a_r):
        h1a = jnp.maximum(_dot(x_r[...], sk0_r[...]) + b0_r[0, :H], 0.0)
        s1a_r[...] = _dot(h1a, sk1_r[0]).astype(jnp.bfloat16)
        t1a_r[...] = _dot(h1a, nk1_r[0]).astype(jnp.bfloat16)

    return pl.pallas_call(
        body,
        grid=(GRID,),
        in_specs=[_rspec(BM, D), _wspec(D, H), _wspec(1, 2 * H),
                  _wspec(2, H, H), _wspec(2, H, H)],
        out_specs=[_rspec(BM, H), _rspec(BM, H)],
        out_shape=[jax.ShapeDtypeStruct((NP, H), jnp.bfloat16),
                   jax.ShapeDtypeStruct((NP, H), jnp.bfloat16)],
    )(x, sk0, b0r, sk1r, nk1r)


def _tc_mid(agg0, degp, t1a, nk0, b0r, nk1r):
    def body(a_r, degp_r, t1a_r, nk0_r, b0_r, nk1_r, t1_r, h1b_r):
        deg = jnp.maximum(jnp.sum(degp_r[...].reshape(NW, BM), axis=0), 1.0)
        mean = (a_r[0] + a_r[1]) / deg[:, None]
        h1b = jnp.maximum(_dot(mean, nk0_r[...]) + b0_r[0, H:], 0.0)
        h1b_r[...] = h1b.astype(jnp.bfloat16)
        t1 = t1a_r[...].astype(_f32) + _dot(h1b, nk1_r[1])
        t1_r[0] = t1[:, :D]
        t1_r[1] = t1[:, D:]

    return pl.pallas_call(
        body,
        grid=(GRID,),
        in_specs=[_spec2, _dspec, _rspec(BM, H),
                  _wspec(D, H), _wspec(1, 2 * H), _wspec(2, H, H)],
        out_specs=[_spec2, _rspec(BM, H)],
        out_shape=[jax.ShapeDtypeStruct((2, NP, D), _f32),
                   jax.ShapeDtypeStruct((NP, H), jnp.bfloat16)],
    )(agg0, degp, t1a, nk0, b0r, nk1r)


def _tc_pre2(s1a, h1b, sk1r, b1r, w1r):
    def body(s1a_r, h1b_r, sk1_r, b1_r, w1_r, p3_r):
        s1 = s1a_r[...].astype(_f32) + _dot(h1b_r[...].astype(_f32), sk1_r[1])
        h2a = jnp.maximum(s1 + b1_r[0, :H], 0.0)
        p3_r[...] = _dot(h2a, w1_r[0]).astype(jnp.bfloat16)

    return pl.pallas_call(
        body,
        grid=(GRID,),
        in_specs=[_rspec(BM, H), _rspec(BM, H), _wspec(2, H, H),
                  _wspec(1, 2 * H), _wspec(2, H, H)],
        out_specs=_rspec(BM, H),
        out_shape=jax.ShapeDtypeStruct((NP, H), jnp.bfloat16),
    )(s1a, h1b, sk1r, b1r, w1r)


def _tc_post(agg1, degp, p3, b1r, w1r, b1mr, w2, b2mr):
    def body(a_r, degp_r, p3_r, b1_r, w1_r, b1m_r, w2_r, b2m_r, o_r):
        deg = jnp.maximum(jnp.sum(degp_r[...].reshape(NW, BM), axis=0), 1.0)
        m = jnp.concatenate([a_r[0], a_r[1]], axis=1) / deg[:, None]
        h2b = jnp.maximum(m + b1_r[0, H:], 0.0)
        h3 = jnp.maximum(p3_r[...].astype(_f32) + _dot(h2b, w1_r[1])
                         + b1m_r[0], 0.0)
        o_r[...] = _dot(h3, w2_r[...]) + b2m_r[0]

    return pl.pallas_call(
        body,
        grid=(GRID,),
        in_specs=[_spec2, _dspec,
                  _rspec(BM, H), _wspec(1, 2 * H), _wspec(2, H, H),
                  _wspec(1, H), _wspec(H, C), _wspec(1, C)],
        out_specs=_rspec(BM, C),
        out_shape=jax.ShapeDtypeStruct((N, C), _f32),
    )(agg1, degp, p3, b1r, w1r, b1mr, w2, b2mr)


def kernel(x, edge_index, edge_weight, self_k0, nbr_k0, b0,
           self_k1, nbr_k1, b1, mlp_w1, mlp_b1, mlp_w2, mlp_b2):
    b0r = b0.reshape(1, 2 * H)
    b1r = b1.reshape(1, 2 * H)
    sk1r = self_k1.reshape(2, H, H)
    nk1r = nbr_k1.reshape(2, H, H)
    w1r = mlp_w1.reshape(2, H, H)
    b1mr = mlp_b1.reshape(1, H)
    b2mr = mlp_b2.reshape(1, C)
    ei4 = edge_index.reshape(2, NW, CPW, CB)

    agg0, degp = _sc_agg(x, ei4, True, split_edges=True, stage=16)
    s1a, t1a = _tc_pre1(x, self_k0, b0r, sk1r, nk1r)

    t1, h1b = _tc_mid(agg0, degp, t1a, nbr_k0, b0r, nk1r)

    (agg1,) = _sc_agg(t1, ei4, False, split_edges=False, stage=40)

    p3 = _tc_pre2(s1a, h1b, sk1r, b1r, w1r)

    return _tc_post(agg1, degp, p3, b1r, w1r, b1mr, mlp_w2, b2mr)


# speedup vs baseline: 1.0160x; 1.0160x over previous
import jax
import jax.numpy as jnp
from jax import lax
from jax.experimental import pallas as pl
from jax.experimental.pallas import tpu as pltpu
from jax.experimental.pallas import tpu_sc as plsc

N = 10000
NP = 10240
E = 320000
D = 128
H = 256
C = 6

NCORE = 2
NSUB = 16
CB = 125
NW = NCORE * NSUB
CPW = E // (NW * CB)
RPT = NP // NSUB
BM = 5120
GRID = NP // BM
STAGE = 16


def _sc_agg(table, ei4, with_deg, *, split_edges, stage):
    mesh = plsc.VectorSubcoreMesh(core_axis_name="c", subcore_axis_name="s")
    out_type = [jax.ShapeDtypeStruct((NCORE, NP, D), jnp.float32)]
    if with_deg:
        out_type.append(jax.ShapeDtypeStruct((NW, 1, NP), jnp.float32))
    spw = CPW // stage
    phases = spw if split_edges else 2 * spw
    scratch = [
        pltpu.VMEM_SHARED((NP, D), jnp.float32),
        pltpu.VMEM((stage, CB), jnp.int32),
        pltpu.VMEM((stage, CB), jnp.int32),
        pltpu.VMEM((CB, D), jnp.float32),
        pltpu.VMEM((CB, D), jnp.float32),
        pltpu.SemaphoreType.DMA,
        pltpu.SemaphoreType.DMA,
    ]
    if with_deg:
        scratch.append(pltpu.VMEM((NP,), jnp.float32))

    def body(*refs):
        if with_deg:
            (table_h, ei_h, agg_o, deg_o,
             acc, row_v, col_v, gbuf0, gbuf1, sem0, sem1, deg_v) = refs
        else:
            (table_h, ei_h, agg_o,
             acc, row_v, col_v, gbuf0, gbuf1, sem0, sem1) = refs
        c = lax.axis_index("c")
        s = lax.axis_index("s")
        tbl = table_h.at[c] if table.ndim == 3 else table_h

        zeros16 = jnp.zeros((16,), jnp.float32)

        def gzero(i, carry):
            gbuf0[i // 8, pl.ds((i % 8) * 16, 16)] = zeros16
            return carry

        lax.fori_loop(0, CB * 8, gzero, 0)
        for q in range(RPT // CB):
            pltpu.sync_copy(gbuf0, acc.at[pl.ds(s * RPT + q * CB, CB)])
        tail = RPT - (RPT // CB) * CB
        if tail:
            pltpu.sync_copy(gbuf0.at[pl.ds(0, tail)],
                            acc.at[pl.ds(s * RPT + (RPT // CB) * CB, tail)])

        if with_deg:
            def dzero(i, carry):
                deg_v[pl.ds(i * 16, 16)] = zeros16
                return carry

            lax.fori_loop(0, NP // 16, dzero, 0)

        plsc.subcore_barrier()

        ones16 = jnp.ones((16,), jnp.float32)
        tailmask = lax.iota(jnp.int32, 16) >= 3

        def dupd(j):
            if with_deg:
                for k in range(7):
                    idx = row_v[j, pl.ds(k * 16, 16)]
                    plsc.addupdate_scatter(deg_v, [idx], ones16)
                idx = row_v[j, pl.ds(CB - 16, 16)]
                plsc.addupdate_scatter(deg_v, [idx], ones16, mask=tailmask)

        for p in range(phases):
            if split_edges:
                wrow = c * NSUB + s
                roff = p * stage
            else:
                wrow = 2 * s + p // spw
                roff = (p % spw) * stage
            pltpu.sync_copy(ei_h.at[0, wrow, pl.ds(roff, stage)], row_v)
            pltpu.sync_copy(ei_h.at[1, wrow, pl.ds(roff, stage)], col_v)

            pltpu.async_copy(tbl.at[col_v.at[0]], gbuf0, sem0)

            def pair(i, carry):
                j0 = 2 * i
                pltpu.async_copy(tbl.at[col_v.at[j0 + 1]], gbuf1, sem1)
                pltpu.make_async_copy(
                    tbl.at[col_v.at[j0]], gbuf0, sem0).wait()
                dupd(j0)
                pltpu.sync_copy(gbuf0, acc.at[row_v.at[j0]], add=True)

                @pl.when(i < stage // 2 - 1)
                def _():
                    pltpu.async_copy(tbl.at[col_v.at[j0 + 2]], gbuf0, sem0)

                pltpu.make_async_copy(
                    tbl.at[col_v.at[j0 + 1]], gbuf1, sem1).wait()
                dupd(j0 + 1)
                pltpu.sync_copy(gbuf1, acc.at[row_v.at[j0 + 1]], add=True)
                return carry

            lax.fori_loop(0, stage // 2, pair, 0)

        if with_deg:
            w = c * NSUB + s
            pltpu.sync_copy(deg_v, deg_o.at[w, 0])

        plsc.subcore_barrier()
        pltpu.sync_copy(acc.at[pl.ds(s * RPT, RPT)],
                        agg_o.at[c, pl.ds(s * RPT, RPT)])

    return pl.kernel(
        body, out_type=out_type, mesh=mesh, scratch_types=scratch,
        compiler_params=pltpu.CompilerParams(needs_layout_passes=False),
    )(table, ei4)


_f32 = jnp.float32


def _dot(a, b):
    return jnp.dot(a, b, preferred_element_type=_f32)


def _wspec(*shape):
    nd = len(shape)
    return pl.BlockSpec(shape, lambda i, nd=nd: (0,) * nd)


def _rspec(*shape):
    return pl.BlockSpec(shape, lambda i: (i,) + (0,) * (len(shape) - 1))


_spec2 = pl.BlockSpec((2, BM, D), lambda i: (0, i, 0))
_dspec = pl.BlockSpec((NW, 1, BM), lambda i: (0, 0, i))


def _tc_pre1(x, sk0, b0r, sk1r, nk1r):
    def body(x_r, sk0_r, b0_r, sk1_r, nk1_r, s1a_r, t1a_r):
        h1a = jnp.maximum(_dot(x_r[...], sk0_r[...]) + b0_r[0, :H], 0.0)
        s1a_r[...] = _dot(h1a, sk1_r[0]).astype(jnp.bfloat16)
        t1a_r[...] = _dot(h1a, nk1_r[0]).astype(jnp.bfloat16)

    return pl.pallas_call(
        body,
        grid=(GRID,),
        in_specs=[_rspec(BM, D), _wspec(D, H), _wspec(1, 2 * H),
                  _wspec(2, H, H), _wspec(2, H, H)],
        out_specs=[_rspec(BM, H), _rspec(BM, H)],
        out_shape=[jax.ShapeDtypeStruct((NP, H), jnp.bfloat16),
                   jax.ShapeDtypeStruct((NP, H), jnp.bfloat16)],
    )(x, sk0, b0r, sk1r, nk1r)


def _tc_mid(agg0, degp, t1a, nk0, b0r, nk1r):
    def body(a_r, degp_r, t1a_r, nk0_r, b0_r, nk1_r, t1_r, h1b_r):
        deg = jnp.maximum(jnp.sum(degp_r[...].reshape(NW, BM), axis=0), 1.0)
        mean = (a_r[0] + a_r[1]) / deg[:, None]
        h1b = jnp.maximum(_dot(mean, nk0_r[...]) + b0_r[0, H:], 0.0)
        h1b_r[...] = h1b.astype(jnp.bfloat16)
        t1 = t1a_r[...].astype(_f32) + _dot(h1b, nk1_r[1])
        t1_r[0] = t1[:, :D]
        t1_r[1] = t1[:, D:]

    return pl.pallas_call(
        body,
        grid=(GRID,),
        in_specs=[_spec2, _dspec, _rspec(BM, H),
                  _wspec(D, H), _wspec(1, 2 * H), _wspec(2, H, H)],
        out_specs=[_spec2, _rspec(BM, H)],
        out_shape=[jax.ShapeDtypeStruct((2, NP, D), _f32),
                   jax.ShapeDtypeStruct((NP, H), jnp.bfloat16)],
    )(agg0, degp, t1a, nk0, b0r, nk1r)


def _tc_pre2(s1a, h1b, sk1r, b1r, w1r):
    def body(s1a_r, h1b_r, sk1_r, b1_r, w1_r, p3_r):
        s1 = s1a_r[...].astype(_f32) + _dot(h1b_r[...].astype(_f32), sk1_r[1])
        h2a = jnp.maximum(s1 + b1_r[0, :H], 0.0)
        p3_r[...] = _dot(h2a, w1_r[0]).astype(jnp.bfloat16)

    return pl.pallas_call(
        body,
        grid=(GRID,),
        in_specs=[_rspec(BM, H), _rspec(BM, H), _wspec(2, H, H),
                  _wspec(1, 2 * H), _wspec(2, H, H)],
        out_specs=_rspec(BM, H),
        out_shape=jax.ShapeDtypeStruct((NP, H), jnp.bfloat16),
    )(s1a, h1b, sk1r, b1r, w1r)


def _tc_post(agg1, degp, p3, b1r, w1r, b1mr, w2, b2mr):
    def body(a_r, degp_r, p3_r, b1_r, w1_r, b1m_r, w2_r, b2m_r, o_r):
        deg = jnp.maximum(jnp.sum(degp_r[...].reshape(NW, BM), axis=0), 1.0)
        m = jnp.concatenate([a_r[0], a_r[1]], axis=1) / deg[:, None]
        h2b = jnp.maximum(m + b1_r[0, H:], 0.0)
        h3 = jnp.maximum(p3_r[...].astype(_f32) + _dot(h2b, w1_r[1])
                         + b1m_r[0], 0.0)
        o_r[...] = _dot(h3, w2_r[...]) + b2m_r[0]

    return pl.pallas_call(
        body,
        grid=(GRID,),
        in_specs=[_spec2, _dspec,
                  _rspec(BM, H), _wspec(1, 2 * H), _wspec(2, H, H),
                  _wspec(1, H), _wspec(H, C), _wspec(1, C)],
        out_specs=_rspec(BM, C),
        out_shape=jax.ShapeDtypeStruct((N, C), _f32),
    )(agg1, degp, p3, b1r, w1r, b1mr, w2, b2mr)


def kernel(x, edge_index, edge_weight, self_k0, nbr_k0, b0,
           self_k1, nbr_k1, b1, mlp_w1, mlp_b1, mlp_w2, mlp_b2):
    b0r = b0.reshape(1, 2 * H)
    b1r = b1.reshape(1, 2 * H)
    sk1r = self_k1.reshape(2, H, H)
    nk1r = nbr_k1.reshape(2, H, H)
    w1r = mlp_w1.reshape(2, H, H)
    b1mr = mlp_b1.reshape(1, H)
    b2mr = mlp_b2.reshape(1, C)
    ei4 = edge_index.reshape(2, NW, CPW, CB)

    agg0, degp = _sc_agg(x, ei4, True, split_edges=True, stage=16)
    s1a, t1a = _tc_pre1(x, self_k0, b0r, sk1r, nk1r)

    t1, h1b = _tc_mid(agg0, degp, t1a, nbr_k0, b0r, nk1r)

    (agg1,) = _sc_agg(t1, ei4, False, split_edges=False, stage=40)

    p3 = _tc_pre2(s1a, h1b, sk1r, b1r, w1r)

    return _tc_post(agg1, degp, p3, b1r, w1r, b1mr, mlp_w2, b2mr)
